# baseline (speedup 1.0000x reference)
import functools
import math

import jax
import jax.numpy as jnp
from jax.experimental import pallas as pl
from jax.experimental.pallas import tpu as pltpu

D_MODEL = 4096
D_HIDDEN = 2048
N_EXPERTS = 64
TOP_K = 8
EPS = 1e-5

TM = 256
HC = 512


def _router_kernel(x_ref, gamma_ref, beta_ref, w1_ref, b1_ref, w2_ref, b2_ref,
                   ew_ref, aux_ref, xn_ref, acc_ref, *, n_tokens, n_steps):
    i = pl.program_id(0)

    @pl.when(i < n_steps)
    def _layernorm():
        xv = x_ref[...]
        mean = jnp.mean(xv, axis=1, keepdims=True)
        xc = xv - mean
        var = jnp.mean(xc * xc, axis=1, keepdims=True)
        xn = xc * jax.lax.rsqrt(var + EPS) * gamma_ref[...] + beta_ref[...]
        xn_ref[i % 2] = xn.astype(jnp.bfloat16)

    @pl.when(i > 0)
    def _mlp_route():
        xn = xn_ref[(i + 1) % 2]
        logits = jnp.broadcast_to(b2_ref[...], (TM, N_EXPERTS))
        for c in range(D_HIDDEN // HC):
            h = jnp.dot(xn, w1_ref[:, c * HC:(c + 1) * HC],
                        preferred_element_type=jnp.float32)
            h = h + b1_ref[:, c * HC:(c + 1) * HC]
            g = 0.5 * h * (1.0 + jax.lax.erf(h * (1.0 / math.sqrt(2.0))))
            logits = logits + jnp.dot(g.astype(jnp.bfloat16),
                                      w2_ref[c * HC:(c + 1) * HC, :],
                                      preferred_element_type=jnp.float32)

        iota = jax.lax.broadcasted_iota(jnp.int32, (TM, N_EXPERTS), 1)
        work = logits
        m0 = jnp.max(work, axis=1, keepdims=True)
        acc = jnp.zeros((TM, N_EXPERTS), dtype=jnp.float32)
        denom = jnp.zeros((TM, 1), dtype=jnp.float32)
        for _ in range(TOP_K):
            m = jnp.max(work, axis=1, keepdims=True)
            idx = jnp.min(jnp.where(work == m, iota, N_EXPERTS), axis=1,
                          keepdims=True)
            oh = iota == idx
            e = jnp.exp(m - m0)
            acc = acc + jnp.where(oh, e, 0.0)
            denom = denom + e
            work = jnp.where(oh, -1e30, work)
        ew = acc / denom
        ew_ref[...] = ew

        @pl.when(i == 1)
        def _init():
            acc_ref[...] = jnp.zeros_like(acc_ref)

        acc_ref[...] += jnp.sum(ew, axis=0, keepdims=True)

        @pl.when(i == n_steps)
        def _finish():
            avg = acc_ref[...] / n_tokens
            d = avg - (1.0 / N_EXPERTS)
            aux_ref[...] = jnp.sum(d * d, keepdims=True).reshape(1, 1)


def kernel(x, gamma, beta, W1, b1, W2, b2):
    B, T, D = x.shape
    n_tokens = B * T
    n_steps = n_tokens // TM
    xf = x.reshape(n_tokens, D)
    last = n_steps - 1

    grid = (n_steps + 1,)
    in_specs = [
        pl.BlockSpec((TM, D), lambda i: (jnp.minimum(i, last), 0)),
        pl.BlockSpec((1, D), lambda i: (0, 0)),
        pl.BlockSpec((1, D), lambda i: (0, 0)),
        pl.BlockSpec((D, D_HIDDEN), lambda i: (0, 0)),
        pl.BlockSpec((1, D_HIDDEN), lambda i: (0, 0)),
        pl.BlockSpec((D_HIDDEN, N_EXPERTS), lambda i: (0, 0)),
        pl.BlockSpec((1, N_EXPERTS), lambda i: (0, 0)),
    ]
    out_specs = [
        pl.BlockSpec((TM, N_EXPERTS), lambda i: (jnp.maximum(i - 1, 0), 0)),
        pl.BlockSpec((1, 1), lambda i: (0, 0)),
    ]

    ew, aux = pl.pallas_call(
        functools.partial(_router_kernel, n_tokens=n_tokens, n_steps=n_steps),
        grid=grid,
        in_specs=in_specs,
        out_specs=out_specs,
        out_shape=[
            jax.ShapeDtypeStruct((n_tokens, N_EXPERTS), jnp.float32),
            jax.ShapeDtypeStruct((1, 1), jnp.float32),
        ],
        scratch_shapes=[
            pltpu.VMEM((2, TM, D), jnp.bfloat16),
            pltpu.VMEM((1, N_EXPERTS), jnp.float32),
        ],
    )(
        xf,
        gamma.reshape(1, D),
        beta.reshape(1, D),
        W1.astype(jnp.bfloat16),
        b1.reshape(1, D_HIDDEN),
        W2.astype(jnp.bfloat16),
        b2.reshape(1, N_EXPERTS),
    )
    return ew.reshape(B, T, N_EXPERTS), aux[0, 0]


# speedup vs baseline: 1.1154x; 1.1154x over previous
import functools
import math

import jax
import jax.numpy as jnp
from jax.experimental import pallas as pl
from jax.experimental.pallas import tpu as pltpu

D_MODEL = 4096
D_HIDDEN = 2048
N_EXPERTS = 64
TOP_K = 8
EPS = 1e-5

TM = 256


def _stage(i, x_ref, gamma_ref, beta_ref, w1_ref, b1_ref, w2_ref, b2_ref,
           ew_ref, aux_ref, wr_ref, rd_ref, acc_ref, n_tokens, n_steps):
    xv = x_ref[...]
    mean = jnp.mean(xv, axis=1, keepdims=True)
    xc = xv - mean
    var = jnp.mean(xc * xc, axis=1, keepdims=True)
    xn = xc * jax.lax.rsqrt(var + EPS) * gamma_ref[...] + beta_ref[...]
    wr_ref[...] = xn.astype(jnp.bfloat16)

    xp = rd_ref[...]
    h = jnp.dot(xp, w1_ref[...], preferred_element_type=jnp.float32)
    h = h + b1_ref[...]
    g = 0.5 * h * (1.0 + jax.lax.erf(h * (1.0 / math.sqrt(2.0))))
    logits = jnp.dot(g.astype(jnp.bfloat16), w2_ref[...],
                     preferred_element_type=jnp.float32)
    logits = logits + b2_ref[...]

    iota = jax.lax.broadcasted_iota(jnp.int32, (TM, N_EXPERTS), 1)
    work = logits
    m0 = jnp.max(work, axis=1, keepdims=True)
    acc = jnp.zeros((TM, N_EXPERTS), dtype=jnp.float32)
    denom = jnp.zeros((TM, 1), dtype=jnp.float32)
    for _ in range(TOP_K):
        m = jnp.max(work, axis=1, keepdims=True)
        idx = jnp.min(jnp.where(work == m, iota, N_EXPERTS), axis=1,
                      keepdims=True)
        oh = iota == idx
        e = jnp.exp(m - m0)
        acc = acc + jnp.where(oh, e, 0.0)
        denom = denom + e
        work = jnp.where(oh, -1e30, work)
    ew = acc / denom
    ew_ref[...] = ew

    @pl.when(i == 1)
    def _init():
        acc_ref[...] = jnp.zeros_like(acc_ref)

    @pl.when(i > 0)
    def _accum():
        acc_ref[...] += jnp.sum(ew, axis=0, keepdims=True)

    @pl.when(i == n_steps)
    def _finish():
        avg = acc_ref[...] / n_tokens
        d = avg - (1.0 / N_EXPERTS)
        aux_ref[...] = jnp.sum(d * d, keepdims=True).reshape(1, 1)


def _router_kernel(x_ref, gamma_ref, beta_ref, w1_ref, b1_ref, w2_ref, b2_ref,
                   ew_ref, aux_ref, xn_a, xn_b, acc_ref, *, n_tokens, n_steps):
    i = pl.program_id(0)
    args = (x_ref, gamma_ref, beta_ref, w1_ref, b1_ref, w2_ref, b2_ref,
            ew_ref, aux_ref)

    @pl.when(i % 2 == 0)
    def _even():
        _stage(i, *args, xn_a, xn_b, acc_ref, n_tokens, n_steps)

    @pl.when(i % 2 == 1)
    def _odd():
        _stage(i, *args, xn_b, xn_a, acc_ref, n_tokens, n_steps)


def kernel(x, gamma, beta, W1, b1, W2, b2):
    B, T, D = x.shape
    n_tokens = B * T
    n_steps = n_tokens // TM
    xf = x.reshape(n_tokens, D)
    last = n_steps - 1

    grid = (n_steps + 1,)
    in_specs = [
        pl.BlockSpec((TM, D), lambda i: (jnp.minimum(i, last), 0)),
        pl.BlockSpec((1, D), lambda i: (0, 0)),
        pl.BlockSpec((1, D), lambda i: (0, 0)),
        pl.BlockSpec((D, D_HIDDEN), lambda i: (0, 0)),
        pl.BlockSpec((1, D_HIDDEN), lambda i: (0, 0)),
        pl.BlockSpec((D_HIDDEN, N_EXPERTS), lambda i: (0, 0)),
        pl.BlockSpec((1, N_EXPERTS), lambda i: (0, 0)),
    ]
    out_specs = [
        pl.BlockSpec((TM, N_EXPERTS), lambda i: (jnp.maximum(i - 1, 0), 0)),
        pl.BlockSpec((1, 1), lambda i: (0, 0)),
    ]

    ew, aux = pl.pallas_call(
        functools.partial(_router_kernel, n_tokens=n_tokens, n_steps=n_steps),
        grid=grid,
        in_specs=in_specs,
        out_specs=out_specs,
        out_shape=[
            jax.ShapeDtypeStruct((n_tokens, N_EXPERTS), jnp.float32),
            jax.ShapeDtypeStruct((1, 1), jnp.float32),
        ],
        scratch_shapes=[
            pltpu.VMEM((TM, D), jnp.bfloat16),
            pltpu.VMEM((TM, D), jnp.bfloat16),
            pltpu.VMEM((1, N_EXPERTS), jnp.float32),
        ],
    )(
        xf,
        gamma.reshape(1, D),
        beta.reshape(1, D),
        W1.astype(jnp.bfloat16),
        b1.reshape(1, D_HIDDEN),
        W2.astype(jnp.bfloat16),
        b2.reshape(1, N_EXPERTS),
    )
    return ew.reshape(B, T, N_EXPERTS), aux[0, 0]


# speedup vs baseline: 1.3051x; 1.1701x over previous
import functools
import math

import jax
import jax.numpy as jnp
from jax.experimental import pallas as pl
from jax.experimental.pallas import tpu as pltpu

D_MODEL = 4096
D_HIDDEN = 2048
N_EXPERTS = 64
TOP_K = 8
EPS = 1e-5

TM = 512


def _router_kernel(x_ref, gamma_ref, beta_ref, w1_ref, b1_ref, w2_ref, b2_ref,
                   ew_ref, aux_ref, acc_ref, *, n_tokens, n_steps):
    i = pl.program_id(0)

    xv = x_ref[...]
    mean = jnp.mean(xv, axis=1, keepdims=True)
    xc = xv - mean
    var = jnp.mean(xc * xc, axis=1, keepdims=True)
    xn = xc * jax.lax.rsqrt(var + EPS) * gamma_ref[...] + beta_ref[...]

    h = jnp.dot(xn.astype(jnp.bfloat16), w1_ref[...],
                preferred_element_type=jnp.float32)
    h = h + b1_ref[...]
    h = 0.5 * h * (1.0 + jax.lax.erf(h * (1.0 / math.sqrt(2.0))))

    logits = jnp.dot(h.astype(jnp.bfloat16), w2_ref[...],
                     preferred_element_type=jnp.float32)
    logits = logits + b2_ref[...]

    iota = jax.lax.broadcasted_iota(jnp.int32, (TM, N_EXPERTS), 1)
    work = logits
    m0 = jnp.max(work, axis=1, keepdims=True)
    acc = jnp.zeros((TM, N_EXPERTS), dtype=jnp.float32)
    denom = jnp.zeros((TM, 1), dtype=jnp.float32)
    for _ in range(TOP_K):
        m = jnp.max(work, axis=1, keepdims=True)
        idx = jnp.min(jnp.where(work == m, iota, N_EXPERTS), axis=1,
                      keepdims=True)
        oh = iota == idx
        e = jnp.exp(m - m0)
        acc = acc + jnp.where(oh, e, 0.0)
        denom = denom + e
        work = jnp.where(oh, -1e30, work)
    ew = acc / denom
    ew_ref[...] = ew

    @pl.when(i == 0)
    def _init():
        acc_ref[...] = jnp.zeros_like(acc_ref)

    acc_ref[...] += jnp.sum(ew, axis=0, keepdims=True)

    @pl.when(i == n_steps - 1)
    def _finish():
        avg = acc_ref[...] / n_tokens
        d = avg - (1.0 / N_EXPERTS)
        aux_ref[...] = jnp.sum(d * d, keepdims=True).reshape(1, 1)


def kernel(x, gamma, beta, W1, b1, W2, b2):
    B, T, D = x.shape
    n_tokens = B * T
    n_steps = n_tokens // TM
    xf = x.reshape(n_tokens, D)

    grid = (n_steps,)
    in_specs = [
        pl.BlockSpec((TM, D), lambda i: (i, 0)),
        pl.BlockSpec((1, D), lambda i: (0, 0)),
        pl.BlockSpec((1, D), lambda i: (0, 0)),
        pl.BlockSpec((D, D_HIDDEN), lambda i: (0, 0)),
        pl.BlockSpec((1, D_HIDDEN), lambda i: (0, 0)),
        pl.BlockSpec((D_HIDDEN, N_EXPERTS), lambda i: (0, 0)),
        pl.BlockSpec((1, N_EXPERTS), lambda i: (0, 0)),
    ]
    out_specs = [
        pl.BlockSpec((TM, N_EXPERTS), lambda i: (i, 0)),
        pl.BlockSpec((1, 1), lambda i: (0, 0)),
    ]

    ew, aux = pl.pallas_call(
        functools.partial(_router_kernel, n_tokens=n_tokens, n_steps=n_steps),
        grid=grid,
        in_specs=in_specs,
        out_specs=out_specs,
        out_shape=[
            jax.ShapeDtypeStruct((n_tokens, N_EXPERTS), jnp.float32),
            jax.ShapeDtypeStruct((1, 1), jnp.float32),
        ],
        scratch_shapes=[pltpu.VMEM((1, N_EXPERTS), jnp.float32)],
        compiler_params=pltpu.CompilerParams(
            vmem_limit_bytes=60 * 1024 * 1024),
    )(
        xf,
        gamma.reshape(1, D),
        beta.reshape(1, D),
        W1.astype(jnp.bfloat16),
        b1.reshape(1, D_HIDDEN),
        W2.astype(jnp.bfloat16),
        b2.reshape(1, N_EXPERTS),
    )
    return ew.reshape(B, T, N_EXPERTS), aux[0, 0]


# speedup vs baseline: 1.3825x; 1.0593x over previous
import functools
import math

import jax
import jax.numpy as jnp
from jax.experimental import pallas as pl
from jax.experimental.pallas import tpu as pltpu

D_MODEL = 4096
D_HIDDEN = 2048
N_EXPERTS = 64
TOP_K = 8
EPS = 1e-5

TM = 1024


def _router_kernel(x_ref, gamma_ref, beta_ref, w1_ref, b1_ref, w2_ref, b2_ref,
                   ew_ref, aux_ref, acc_ref, *, n_tokens, n_steps):
    i = pl.program_id(0)

    xv = x_ref[...]
    s1 = jnp.sum(xv, axis=1, keepdims=True)
    s2 = jnp.sum(xv * xv, axis=1, keepdims=True)
    mean = s1 * (1.0 / D_MODEL)
    var = s2 * (1.0 / D_MODEL) - mean * mean
    r = jax.lax.rsqrt(var + EPS)
    xn = (xv - mean) * r * gamma_ref[...] + beta_ref[...]

    h = jnp.dot(xn.astype(jnp.bfloat16), w1_ref[...],
                preferred_element_type=jnp.float32)
    h = h + b1_ref[...]
    h = 0.5 * h * (1.0 + jax.lax.erf(h * (1.0 / math.sqrt(2.0))))

    logits = jnp.dot(h.astype(jnp.bfloat16), w2_ref[...],
                     preferred_element_type=jnp.float32)
    logits = logits + b2_ref[...]

    iota = jax.lax.broadcasted_iota(jnp.int32, (TM, N_EXPERTS), 1)
    work = logits
    m0 = jnp.max(work, axis=1, keepdims=True)
    acc = jnp.zeros((TM, N_EXPERTS), dtype=jnp.float32)
    denom = jnp.zeros((TM, 1), dtype=jnp.float32)
    for _ in range(TOP_K):
        m = jnp.max(work, axis=1, keepdims=True)
        idx = jnp.min(jnp.where(work == m, iota, N_EXPERTS), axis=1,
                      keepdims=True)
        oh = iota == idx
        e = jnp.exp(m - m0)
        acc = acc + jnp.where(oh, e, 0.0)
        denom = denom + e
        work = jnp.where(oh, -1e30, work)
    ew = acc / denom
    ew_ref[...] = ew

    @pl.when(i == 0)
    def _init():
        acc_ref[...] = jnp.zeros_like(acc_ref)

    acc_ref[...] += jnp.sum(ew, axis=0, keepdims=True)

    @pl.when(i == n_steps - 1)
    def _finish():
        avg = acc_ref[...] / n_tokens
        d = avg - (1.0 / N_EXPERTS)
        aux_ref[...] = jnp.sum(d * d, keepdims=True).reshape(1, 1)


def kernel(x, gamma, beta, W1, b1, W2, b2):
    B, T, D = x.shape
    n_tokens = B * T
    n_steps = n_tokens // TM
    xf = x.reshape(n_tokens, D)

    grid = (n_steps,)
    in_specs = [
        pl.BlockSpec((TM, D), lambda i: (i, 0)),
        pl.BlockSpec((1, D), lambda i: (0, 0)),
        pl.BlockSpec((1, D), lambda i: (0, 0)),
        pl.BlockSpec((D, D_HIDDEN), lambda i: (0, 0)),
        pl.BlockSpec((1, D_HIDDEN), lambda i: (0, 0)),
        pl.BlockSpec((D_HIDDEN, N_EXPERTS), lambda i: (0, 0)),
        pl.BlockSpec((1, N_EXPERTS), lambda i: (0, 0)),
    ]
    out_specs = [
        pl.BlockSpec((TM, N_EXPERTS), lambda i: (i, 0)),
        pl.BlockSpec((1, 1), lambda i: (0, 0)),
    ]

    ew, aux = pl.pallas_call(
        functools.partial(_router_kernel, n_tokens=n_tokens, n_steps=n_steps),
        grid=grid,
        in_specs=in_specs,
        out_specs=out_specs,
        out_shape=[
            jax.ShapeDtypeStruct((n_tokens, N_EXPERTS), jnp.float32),
            jax.ShapeDtypeStruct((1, 1), jnp.float32),
        ],
        scratch_shapes=[pltpu.VMEM((1, N_EXPERTS), jnp.float32)],
        compiler_params=pltpu.CompilerParams(
            vmem_limit_bytes=100 * 1024 * 1024),
    )(
        xf,
        gamma.reshape(1, D),
        beta.reshape(1, D),
        W1.astype(jnp.bfloat16),
        b1.reshape(1, D_HIDDEN),
        W2.astype(jnp.bfloat16),
        b2.reshape(1, N_EXPERTS),
    )
    return ew.reshape(B, T, N_EXPERTS), aux[0, 0]


# speedup vs baseline: 1.4493x; 1.0483x over previous
import functools
import math

import jax
import jax.numpy as jnp
from jax.experimental import pallas as pl
from jax.experimental.pallas import tpu as pltpu

D_MODEL = 4096
D_HIDDEN = 2048
N_EXPERTS = 64
TOP_K = 8
EPS = 1e-5

TM = 1024


NSUB = 2
SM = TM // NSUB


def _subtile(xv, gamma, beta, w1, b1, w2, b2):
    s1 = jnp.sum(xv, axis=1, keepdims=True)
    s2 = jnp.sum(xv * xv, axis=1, keepdims=True)
    mean = s1 * (1.0 / D_MODEL)
    var = s2 * (1.0 / D_MODEL) - mean * mean
    r = jax.lax.rsqrt(var + EPS)
    xn = (xv - mean) * r * gamma + beta

    h = jnp.dot(xn.astype(jnp.bfloat16), w1,
                preferred_element_type=jnp.float32)
    h = h + b1
    h = 0.5 * h * (1.0 + jax.lax.erf(h * (1.0 / math.sqrt(2.0))))

    logits = jnp.dot(h.astype(jnp.bfloat16), w2,
                     preferred_element_type=jnp.float32)
    logits = logits + b2

    iota = jax.lax.broadcasted_iota(jnp.int32, (SM, N_EXPERTS), 1)
    work = logits
    m0 = jnp.max(work, axis=1, keepdims=True)
    acc = jnp.zeros((SM, N_EXPERTS), dtype=jnp.float32)
    denom = jnp.zeros((SM, 1), dtype=jnp.float32)
    for _ in range(TOP_K):
        m = jnp.max(work, axis=1, keepdims=True)
        idx = jnp.min(jnp.where(work == m, iota, N_EXPERTS), axis=1,
                      keepdims=True)
        oh = iota == idx
        e = jnp.exp(m - m0)
        acc = acc + jnp.where(oh, e, 0.0)
        denom = denom + e
        work = jnp.where(oh, -1e30, work)
    return acc / denom


def _router_kernel(x_ref, gamma_ref, beta_ref, w1_ref, b1_ref, w2_ref, b2_ref,
                   ew_ref, aux_ref, acc_ref, *, n_tokens, n_steps):
    i = pl.program_id(0)

    esums = []
    for s in range(NSUB):
        ew = _subtile(x_ref[s * SM:(s + 1) * SM, :], gamma_ref[...],
                      beta_ref[...], w1_ref[...], b1_ref[...], w2_ref[...],
                      b2_ref[...])
        ew_ref[s * SM:(s + 1) * SM, :] = ew
        esums.append(jnp.sum(ew, axis=0, keepdims=True))

    @pl.when(i == 0)
    def _init():
        acc_ref[...] = jnp.zeros_like(acc_ref)

    acc_ref[...] += sum(esums)

    @pl.when(i == n_steps - 1)
    def _finish():
        avg = acc_ref[...] / n_tokens
        d = avg - (1.0 / N_EXPERTS)
        aux_ref[...] = jnp.sum(d * d, keepdims=True).reshape(1, 1)


def kernel(x, gamma, beta, W1, b1, W2, b2):
    B, T, D = x.shape
    n_tokens = B * T
    n_steps = n_tokens // TM
    xf = x.reshape(n_tokens, D)

    grid = (n_steps,)
    in_specs = [
        pl.BlockSpec((TM, D), lambda i: (i, 0)),
        pl.BlockSpec((1, D), lambda i: (0, 0)),
        pl.BlockSpec((1, D), lambda i: (0, 0)),
        pl.BlockSpec((D, D_HIDDEN), lambda i: (0, 0)),
        pl.BlockSpec((1, D_HIDDEN), lambda i: (0, 0)),
        pl.BlockSpec((D_HIDDEN, N_EXPERTS), lambda i: (0, 0)),
        pl.BlockSpec((1, N_EXPERTS), lambda i: (0, 0)),
    ]
    out_specs = [
        pl.BlockSpec((TM, N_EXPERTS), lambda i: (i, 0)),
        pl.BlockSpec((1, 1), lambda i: (0, 0)),
    ]

    ew, aux = pl.pallas_call(
        functools.partial(_router_kernel, n_tokens=n_tokens, n_steps=n_steps),
        grid=grid,
        in_specs=in_specs,
        out_specs=out_specs,
        out_shape=[
            jax.ShapeDtypeStruct((n_tokens, N_EXPERTS), jnp.float32),
            jax.ShapeDtypeStruct((1, 1), jnp.float32),
        ],
        scratch_shapes=[pltpu.VMEM((1, N_EXPERTS), jnp.float32)],
        compiler_params=pltpu.CompilerParams(
            vmem_limit_bytes=100 * 1024 * 1024),
    )(
        xf,
        gamma.reshape(1, D),
        beta.reshape(1, D),
        W1.astype(jnp.bfloat16),
        b1.reshape(1, D_HIDDEN),
        W2.astype(jnp.bfloat16),
        b2.reshape(1, N_EXPERTS),
    )
    return ew.reshape(B, T, N_EXPERTS), aux[0, 0]
